# Initial kernel scaffold; baseline (speedup 1.0000x reference)
import jax
import jax.numpy as jnp
from jax.experimental import pallas as pl


def kernel(z, edge_index):
    raise NotImplementedError("write your pallas kernel here")


# speedup vs baseline: 4.0973x; 4.0973x over previous
import functools

import jax
import jax.numpy as jnp
from jax import lax
from jax.experimental import pallas as pl
from jax.experimental.pallas import tpu as pltpu
from jax.experimental.pallas import tpu_sc as plsc

E = 320_000
V = 10_000
D = 128
NC = 2
NS = 16
NW = NC * NS
EW = E // NW
C = 80
NCHUNK = EW // C
L = 16


def _dot16(srows, drows, slot, e):
    acc = srows[slot, e, pl.ds(0, L)] * drows[slot, e, pl.ds(0, L)]
    for k in range(1, D // L):
        acc += srows[slot, e, pl.ds(k * L, L)] * drows[slot, e, pl.ds(k * L, L)]
    return jnp.sum(acc)


def _sc_body(z_hbm, sidx_hbm, didx_hbm, out_hbm,
             sidx_v, didx_v, srows, drows, ovals, gsem):
    wid = lax.axis_index("s") * NC + lax.axis_index("c")
    base = pl.multiple_of(wid * EW, 8)

    pltpu.sync_copy(sidx_hbm.at[pl.ds(base, EW)], sidx_v)
    pltpu.sync_copy(didx_hbm.at[pl.ds(base, EW)], didx_v)

    def issue_gather(c, slot):
        off = pl.multiple_of(c * C, 8)
        pltpu.async_copy(z_hbm.at[sidx_v.at[pl.ds(off, C)]],
                         srows.at[slot], gsem.at[slot])
        pltpu.async_copy(z_hbm.at[didx_v.at[pl.ds(off, C)]],
                         drows.at[slot], gsem.at[slot])

    def wait_gather(c, slot):
        off = pl.multiple_of(c * C, 8)
        pltpu.make_async_copy(z_hbm.at[sidx_v.at[pl.ds(off, C)]],
                              srows.at[slot], gsem.at[slot]).wait()
        pltpu.make_async_copy(z_hbm.at[didx_v.at[pl.ds(off, C)]],
                              drows.at[slot], gsem.at[slot]).wait()

    def compute(c, slot):
        obase = c * C
        lanes = lax.iota(jnp.int32, L)

        def group(g, _):
            res = jnp.zeros((L,), jnp.float32)
            for e16 in range(L):
                e = g * L + e16
                res = jnp.where(lanes == e16, _dot16(srows, drows, slot, e),
                                res)
            off = pl.multiple_of(obase, 8) + g * L
            ovals[pl.ds(off, L)] = 1.0 / (1.0 + jnp.exp(-res))
            return 0

        lax.fori_loop(0, C // L, group, 0)

    issue_gather(0, 0)

    def body(j, _):
        a = 2 * j
        b = a + 1
        issue_gather(b, 1)
        wait_gather(a, 0)
        compute(a, 0)
        issue_gather(b + 1, 0)
        wait_gather(b, 1)
        compute(b, 1)
        return 0

    lax.fori_loop(0, (NCHUNK - 1) // 2, body, 0)
    wait_gather(NCHUNK - 1, 0)
    compute(NCHUNK - 1, 0)

    pltpu.sync_copy(ovals, out_hbm.at[pl.ds(base, EW)])


@jax.jit
def _decode(z, src_idx, dst_idx):
    mesh = plsc.VectorSubcoreMesh(
        core_axis_name="c", subcore_axis_name="s",
        num_cores=NC, num_subcores=NS,
    )
    return pl.kernel(
        _sc_body,
        out_type=jax.ShapeDtypeStruct((E,), jnp.float32),
        mesh=mesh,
        scratch_types=[
            pltpu.VMEM((EW,), jnp.int32),
            pltpu.VMEM((EW,), jnp.int32),
            pltpu.VMEM((2, C, D), jnp.float32),
            pltpu.VMEM((2, C, D), jnp.float32),
            pltpu.VMEM((EW,), jnp.float32),
            pltpu.SemaphoreType.DMA((2,)),
        ],
        compiler_params=pltpu.CompilerParams(needs_layout_passes=False),
    )(z, src_idx, dst_idx)


def kernel(z, edge_index):
    src_idx = edge_index[0].astype(jnp.int32)
    dst_idx = edge_index[1].astype(jnp.int32)
    return _decode(z, src_idx, dst_idx)


# speedup vs baseline: 9.7784x; 2.3866x over previous
import functools

import jax
import jax.numpy as jnp
from jax import lax
from jax.experimental import pallas as pl
from jax.experimental.pallas import tpu as pltpu
from jax.experimental.pallas import tpu_sc as plsc

E = 320_000
V = 10_000
D = 128
NC = 2
NS = 16
NW = NC * NS
EW = E // NW
C = 80
NCHUNK = EW // C
L = 16


def _dot16(srows, drows, slot, e):
    acc = None
    for k in range(D // (2 * L)):
        s = plsc.bitcast(srows[slot, e, pl.ds(k * L, L)], jnp.bfloat16)
        d = plsc.bitcast(drows[slot, e, pl.ds(k * L, L)], jnp.bfloat16)
        p0, p1 = plsc.unpack(s * d, format=plsc.PackFormat.INTERLEAVED,
                             preferred_element_type=jnp.float32)
        acc = p0 + p1 if acc is None else acc + p0 + p1
    return jnp.sum(acc)


def _sc_body(z_hbm, sidx_hbm, didx_hbm, out_hbm,
             sidx_v, didx_v, srows, drows, ovals, gsem):
    wid = lax.axis_index("s") * NC + lax.axis_index("c")
    base = pl.multiple_of(wid * EW, 8)

    pltpu.sync_copy(sidx_hbm.at[pl.ds(base, EW)], sidx_v)
    pltpu.sync_copy(didx_hbm.at[pl.ds(base, EW)], didx_v)

    def issue_gather(c, slot):
        off = pl.multiple_of(c * C, 8)
        pltpu.async_copy(z_hbm.at[sidx_v.at[pl.ds(off, C)]],
                         srows.at[slot], gsem.at[slot])
        pltpu.async_copy(z_hbm.at[didx_v.at[pl.ds(off, C)]],
                         drows.at[slot], gsem.at[slot])

    def wait_gather(c, slot):
        off = pl.multiple_of(c * C, 8)
        pltpu.make_async_copy(z_hbm.at[sidx_v.at[pl.ds(off, C)]],
                              srows.at[slot], gsem.at[slot]).wait()
        pltpu.make_async_copy(z_hbm.at[didx_v.at[pl.ds(off, C)]],
                              drows.at[slot], gsem.at[slot]).wait()

    def compute(c, slot):
        obase = c * C
        lanes = lax.iota(jnp.int32, L)

        def group(g, _):
            res = jnp.zeros((L,), jnp.float32)
            for e16 in range(L):
                e = g * L + e16
                res = jnp.where(lanes == e16, _dot16(srows, drows, slot, e),
                                res)
            off = pl.multiple_of(obase, 8) + g * L
            ovals[pl.ds(off, L)] = 1.0 / (1.0 + jnp.exp(-res))
            return 0

        lax.fori_loop(0, C // L, group, 0)

    issue_gather(0, 0)

    def body(j, _):
        a = 2 * j
        b = a + 1
        issue_gather(b, 1)
        wait_gather(a, 0)
        compute(a, 0)
        issue_gather(b + 1, 0)
        wait_gather(b, 1)
        compute(b, 1)
        return 0

    lax.fori_loop(0, (NCHUNK - 1) // 2, body, 0)
    wait_gather(NCHUNK - 1, 0)
    compute(NCHUNK - 1, 0)

    pltpu.sync_copy(ovals, out_hbm.at[pl.ds(base, EW)])


@jax.jit
def _decode(z, src_idx, dst_idx):
    mesh = plsc.VectorSubcoreMesh(
        core_axis_name="c", subcore_axis_name="s",
        num_cores=NC, num_subcores=NS,
    )
    return pl.kernel(
        _sc_body,
        out_type=jax.ShapeDtypeStruct((E,), jnp.float32),
        mesh=mesh,
        scratch_types=[
            pltpu.VMEM((EW,), jnp.int32),
            pltpu.VMEM((EW,), jnp.int32),
            pltpu.VMEM((2, C, D // 2), jnp.int32),
            pltpu.VMEM((2, C, D // 2), jnp.int32),
            pltpu.VMEM((EW,), jnp.float32),
            pltpu.SemaphoreType.DMA((2,)),
        ],
        compiler_params=pltpu.CompilerParams(needs_layout_passes=False,
                                             use_tc_tiling_on_sc=False),
    )(z, src_idx, dst_idx)


def kernel(z, edge_index):
    src_idx = edge_index[0].astype(jnp.int32)
    dst_idx = edge_index[1].astype(jnp.int32)
    zb = jax.lax.bitcast_convert_type(
        z.astype(jnp.bfloat16).reshape(V, D // 2, 2), jnp.int32)
    return _decode(zb, src_idx, dst_idx)
